# Initial kernel scaffold; baseline (speedup 1.0000x reference)
import jax
import jax.numpy as jnp
from jax.experimental import pallas as pl


def kernel(x, pos_embedding):
    raise NotImplementedError("write your pallas kernel here")


# speedup vs baseline: 3.4158x; 3.4158x over previous
import jax
import jax.numpy as jnp
from jax.experimental import pallas as pl


_BLOCK_S = 512


def _copy_body(emb_ref, out_ref):
    out_ref[...] = emb_ref[...][None]


def kernel(x, pos_embedding):
    batch, seq_len = x.shape
    max_len, d_model = pos_embedding.shape
    num_s_blocks = seq_len // _BLOCK_S

    out = pl.pallas_call(
        _copy_body,
        grid=(num_s_blocks, batch),
        in_specs=[
            pl.BlockSpec((_BLOCK_S, d_model), lambda s, b: (s, 0)),
        ],
        out_specs=pl.BlockSpec((1, _BLOCK_S, d_model), lambda s, b: (b, s, 0)),
        out_shape=jax.ShapeDtypeStruct((batch, seq_len, d_model), jnp.float32),
    )(pos_embedding)
    return out


# speedup vs baseline: 3.5487x; 1.0389x over previous
import functools

import jax
import jax.numpy as jnp
from jax import lax
from jax.experimental import pallas as pl
from jax.experimental.pallas import tpu as pltpu
from jax.experimental.pallas import tpu_sc as plsc

_NC = 2
_NS = 16
_NW = _NC * _NS
_CH = 32


def _make_sc_copy(batch, seq_len, d_model):
    rows_per_w = seq_len // _NW
    nch = rows_per_w // _CH
    mesh = plsc.VectorSubcoreMesh(core_axis_name="c", subcore_axis_name="s")

    @functools.partial(
        pl.kernel,
        mesh=mesh,
        out_type=jax.ShapeDtypeStruct((batch * seq_len, d_model), jnp.float32),
        scratch_types=[
            pltpu.VMEM((_CH, d_model), jnp.float32),
            pltpu.VMEM((_CH, d_model), jnp.float32),
            pltpu.SemaphoreType.DMA,
            pltpu.SemaphoreType.DMA,
        ],
    )
    def sc_copy(table_hbm, out_hbm, buf0, buf1, insem, outsem):
        wid = lax.axis_index("s") * _NC + lax.axis_index("c")
        s0 = wid * rows_per_w
        bufs = (buf0, buf1)
        in_h = [None] * nch
        out_h = [None] * nch
        in_h[0] = pltpu.async_copy(table_hbm.at[pl.ds(s0, _CH)], buf0, insem)
        for c in range(nch):
            if c >= 1:
                for h in out_h[c - 1]:
                    h.wait()
            if c + 1 < nch:
                in_h[c + 1] = pltpu.async_copy(
                    table_hbm.at[pl.ds(s0 + (c + 1) * _CH, _CH)],
                    bufs[(c + 1) % 2],
                    insem,
                )
            in_h[c].wait()
            buf = bufs[c % 2]
            out_h[c] = [
                pltpu.async_copy(
                    buf,
                    out_hbm.at[pl.ds(b * seq_len + s0 + c * _CH, _CH)],
                    outsem,
                )
                for b in range(batch)
            ]
        for h in out_h[nch - 1]:
            h.wait()

    return sc_copy


def kernel(x, pos_embedding):
    batch, seq_len = x.shape
    max_len, d_model = pos_embedding.shape
    out_flat = _make_sc_copy(batch, seq_len, d_model)(pos_embedding)
    return out_flat.reshape(batch, seq_len, d_model)


# speedup vs baseline: 4.8659x; 1.3712x over previous
import jax
import jax.numpy as jnp
from jax.experimental import pallas as pl
from jax.experimental.pallas import tpu as pltpu

_CH = 1024


def _make_tc_copy(batch, seq_len, d_model):
    nch = seq_len // _CH

    def body(emb_hbm, out_hbm, buf0, buf1, insem, outsem):
        bufs = (buf0, buf1)
        in_h = [None] * nch
        out_h = [None] * nch
        in_h[0] = pltpu.make_async_copy(emb_hbm.at[pl.ds(0, _CH)], buf0, insem)
        in_h[0].start()
        for c in range(nch):
            if c >= 1:
                for h in out_h[c - 1]:
                    h.wait()
            if c + 1 < nch:
                in_h[c + 1] = pltpu.make_async_copy(
                    emb_hbm.at[pl.ds((c + 1) * _CH, _CH)], bufs[(c + 1) % 2], insem
                )
                in_h[c + 1].start()
            in_h[c].wait()
            buf = bufs[c % 2]
            out_h[c] = []
            for b in range(batch):
                h = pltpu.make_async_copy(
                    buf, out_hbm.at[pl.ds(b * seq_len + c * _CH, _CH)], outsem
                )
                h.start()
                out_h[c].append(h)
        for h in out_h[nch - 1]:
            h.wait()

    return pl.pallas_call(
        body,
        in_specs=[pl.BlockSpec(memory_space=pl.ANY)],
        out_specs=pl.BlockSpec(memory_space=pl.ANY),
        out_shape=jax.ShapeDtypeStruct((batch * seq_len, d_model), jnp.float32),
        scratch_shapes=[
            pltpu.VMEM((_CH, d_model), jnp.float32),
            pltpu.VMEM((_CH, d_model), jnp.float32),
            pltpu.SemaphoreType.DMA,
            pltpu.SemaphoreType.DMA,
        ],
    )


def kernel(x, pos_embedding):
    batch, seq_len = x.shape
    max_len, d_model = pos_embedding.shape
    out_flat = _make_tc_copy(batch, seq_len, d_model)(pos_embedding)
    return out_flat.reshape(batch, seq_len, d_model)


# speedup vs baseline: 5.1105x; 1.0503x over previous
import jax
import jax.numpy as jnp
from jax.experimental import pallas as pl
from jax.experimental.pallas import tpu as pltpu

_CH = 512
_NBUF = 4


def _make_tc_copy(batch, seq_len, d_model):
    nch = seq_len // _CH

    def body(emb_hbm, out_hbm, *rest):
        bufs = rest[:_NBUF]
        insem, outsem = rest[_NBUF], rest[_NBUF + 1]
        in_h = [None] * nch
        out_h = [None] * nch
        in_h[0] = pltpu.make_async_copy(emb_hbm.at[pl.ds(0, _CH)], bufs[0], insem)
        in_h[0].start()
        for c in range(nch):
            if c + 1 < nch:
                if c + 1 - _NBUF >= 0:
                    for h in out_h[c + 1 - _NBUF]:
                        h.wait()
                in_h[c + 1] = pltpu.make_async_copy(
                    emb_hbm.at[pl.ds((c + 1) * _CH, _CH)],
                    bufs[(c + 1) % _NBUF],
                    insem,
                )
                in_h[c + 1].start()
            in_h[c].wait()
            buf = bufs[c % _NBUF]
            out_h[c] = []
            for b in range(batch):
                h = pltpu.make_async_copy(
                    buf, out_hbm.at[pl.ds(b * seq_len + c * _CH, _CH)], outsem
                )
                h.start()
                out_h[c].append(h)
        for c in range(max(0, nch - _NBUF), nch):
            for h in out_h[c]:
                h.wait()

    return pl.pallas_call(
        body,
        in_specs=[pl.BlockSpec(memory_space=pl.ANY)],
        out_specs=pl.BlockSpec(memory_space=pl.ANY),
        out_shape=jax.ShapeDtypeStruct((batch * seq_len, d_model), jnp.float32),
        scratch_shapes=[pltpu.VMEM((_CH, d_model), jnp.float32) for _ in range(_NBUF)]
        + [pltpu.SemaphoreType.DMA, pltpu.SemaphoreType.DMA],
    )


def kernel(x, pos_embedding):
    batch, seq_len = x.shape
    max_len, d_model = pos_embedding.shape
    out_flat = _make_tc_copy(batch, seq_len, d_model)(pos_embedding)
    return out_flat.reshape(batch, seq_len, d_model)


# speedup vs baseline: 5.1595x; 1.0096x over previous
import jax
import jax.numpy as jnp
from jax.experimental import pallas as pl
from jax.experimental.pallas import tpu as pltpu

_CH = 512
_NBUF = 8


def _make_tc_copy(batch, seq_len, d_model):
    nch = seq_len // _CH

    def body(emb_hbm, out_hbm, *rest):
        bufs = rest[:_NBUF]
        insem, outsem = rest[_NBUF], rest[_NBUF + 1]
        in_h = [None] * nch
        out_h = [None] * nch
        in_h[0] = pltpu.make_async_copy(emb_hbm.at[pl.ds(0, _CH)], bufs[0], insem)
        in_h[0].start()
        for c in range(nch):
            if c + 1 < nch:
                if c + 1 - _NBUF >= 0:
                    for h in out_h[c + 1 - _NBUF]:
                        h.wait()
                in_h[c + 1] = pltpu.make_async_copy(
                    emb_hbm.at[pl.ds((c + 1) * _CH, _CH)],
                    bufs[(c + 1) % _NBUF],
                    insem,
                )
                in_h[c + 1].start()
            in_h[c].wait()
            buf = bufs[c % _NBUF]
            out_h[c] = []
            for b in range(batch):
                h = pltpu.make_async_copy(
                    buf, out_hbm.at[pl.ds(b * seq_len + c * _CH, _CH)], outsem
                )
                h.start()
                out_h[c].append(h)
        for c in range(max(0, nch - _NBUF), nch):
            for h in out_h[c]:
                h.wait()

    return pl.pallas_call(
        body,
        in_specs=[pl.BlockSpec(memory_space=pl.ANY)],
        out_specs=pl.BlockSpec(memory_space=pl.ANY),
        out_shape=jax.ShapeDtypeStruct((batch * seq_len, d_model), jnp.float32),
        scratch_shapes=[pltpu.VMEM((_CH, d_model), jnp.float32) for _ in range(_NBUF)]
        + [pltpu.SemaphoreType.DMA, pltpu.SemaphoreType.DMA],
    )


def kernel(x, pos_embedding):
    batch, seq_len = x.shape
    max_len, d_model = pos_embedding.shape
    out_flat = _make_tc_copy(batch, seq_len, d_model)(pos_embedding)
    return out_flat.reshape(batch, seq_len, d_model)
